# baseline (speedup 1.0000x reference)
import functools

import jax
import jax.numpy as jnp
from jax.experimental import pallas as pl
from jax.experimental.pallas import tpu as pltpu


def _body(N, D, R, K, f_ref, m_ref, t_ref, o_ref, in_buf, out_buf, in_sem, out_sem):
    steps = N // R

    def in_dma(chunk, slot):
        return pltpu.make_async_copy(
            f_ref.at[pl.ds(chunk * R, R), :], in_buf.at[slot], in_sem.at[slot]
        )

    def out_dma(chunk, slot):
        return pltpu.make_async_copy(
            out_buf.at[slot], o_ref.at[pl.ds(chunk * R, R), :], out_sem.at[slot]
        )

    for j in range(K):
        in_dma(j, j).start()

    def step(i, carry):
        slot = jax.lax.rem(i, K)
        in_dma(i, slot).wait()

        @pl.when(i >= K)
        def _():
            out_dma(i - K, slot).wait()

        m = m_ref[:, pl.ds(i * R, R)].astype(jnp.int32).reshape(R, 1) != 0
        out_buf[slot] = jnp.where(m, t_ref[...], in_buf[slot])
        out_dma(i, slot).start()

        @pl.when(i + K < steps)
        def _():
            in_dma(i + K, slot).start()

        return carry

    jax.lax.fori_loop(0, steps, step, 0)
    for j in range(steps - K, steps):
        out_dma(j, j % K).wait()


def kernel(features, mask, mask_token):
    B, S, D = features.shape
    N = B * S
    R = 1024
    K = 4
    f2 = features.reshape(N, D)
    m2 = mask.reshape(1, N)
    t2 = mask_token.reshape(1, D)
    out = pl.pallas_call(
        functools.partial(_body, N, D, R, K),
        in_specs=[
            pl.BlockSpec(memory_space=pl.ANY),
            pl.BlockSpec(memory_space=pltpu.VMEM),
            pl.BlockSpec(memory_space=pltpu.VMEM),
        ],
        out_specs=pl.BlockSpec(memory_space=pl.ANY),
        out_shape=jax.ShapeDtypeStruct((N, D), features.dtype),
        scratch_shapes=[
            pltpu.VMEM((K, R, D), features.dtype),
            pltpu.VMEM((K, R, D), features.dtype),
            pltpu.SemaphoreType.DMA((K,)),
            pltpu.SemaphoreType.DMA((K,)),
        ],
    )(f2, m2, t2)
    return out.reshape(B, S, D)
